# Initial kernel scaffold; baseline (speedup 1.0000x reference)
import jax
import jax.numpy as jnp
from jax.experimental import pallas as pl


def kernel(x, edge_index, W_lin, b_lin, weight, w_ih, w_hh, b_ih, b_hh):
    raise NotImplementedError("write your pallas kernel here")


# speedup vs baseline: 3.3438x; 3.3438x over previous
import functools

import jax
import jax.numpy as jnp
from jax import lax
from jax.experimental import pallas as pl
from jax.experimental.pallas import tpu as pltpu
from jax.experimental.pallas import tpu_sc as plsc

N = 10000
NP = 10240
E = 160000
D = 256
H = D // 2
G = 3 * D
NLAYERS = 3

R = 1000
NS = 16
EPW = E // NS
K = 80
NCHUNK = EPW // K
RPS = NP // NS
ZR = 160
LANES = 16


def _lin_body(x_ref, wT_ref, b_ref, out_ref):
    out_ref[...] = (
        jnp.dot(x_ref[...], wT_ref[...], preferred_element_type=jnp.float32)
        + b_ref[...]
    )


_lin = pl.pallas_call(
    _lin_body,
    grid=(N // R,),
    in_specs=[
        pl.BlockSpec((R, D), lambda i: (i, 0)),
        pl.BlockSpec((D, D), lambda i: (0, 0)),
        pl.BlockSpec((1, D), lambda i: (0, 0)),
    ],
    out_specs=pl.BlockSpec((R, D), lambda i: (i, 0)),
    out_shape=jax.ShapeDtypeStruct((N, D), jnp.float32),
)


def _msg_body(h_ref, wm_ref, whhT_ref, bhh_ref, m_ref, gh_ref):
    h = h_ref[...]
    m = jnp.dot(h, wm_ref[...], preferred_element_type=jnp.float32)
    m_ref[0] = m[:, :H]
    m_ref[1] = m[:, H:]
    gh_ref[...] = (
        jnp.dot(h, whhT_ref[...], preferred_element_type=jnp.float32)
        + bhh_ref[...]
    )


_msg = pl.pallas_call(
    _msg_body,
    grid=(N // R,),
    in_specs=[
        pl.BlockSpec((R, D), lambda i: (i, 0)),
        pl.BlockSpec((D, D), lambda i: (0, 0)),
        pl.BlockSpec((D, G), lambda i: (0, 0)),
        pl.BlockSpec((1, G), lambda i: (0, 0)),
    ],
    out_specs=[
        pl.BlockSpec((2, R, H), lambda i: (0, i, 0)),
        pl.BlockSpec((R, G), lambda i: (i, 0)),
    ],
    out_shape=[
        jax.ShapeDtypeStruct((2, NP, H), jnp.float32),
        jax.ShapeDtypeStruct((N, G), jnp.float32),
    ],
)


def _gru_body(agg_ref, gh_ref, h_ref, wihT_ref, bih_ref, out_ref):
    gi = (
        jnp.dot(agg_ref[0], wihT_ref[:H], preferred_element_type=jnp.float32)
        + jnp.dot(agg_ref[1], wihT_ref[H:], preferred_element_type=jnp.float32)
        + bih_ref[...]
    )
    gh = gh_ref[...]
    h = h_ref[...]
    r = jax.nn.sigmoid(gi[:, :D] + gh[:, :D])
    z = jax.nn.sigmoid(gi[:, D:2 * D] + gh[:, D:2 * D])
    n = jnp.tanh(gi[:, 2 * D:] + r * gh[:, 2 * D:])
    out_ref[...] = (1.0 - z) * n + z * h


_gru = pl.pallas_call(
    _gru_body,
    grid=(N // R,),
    in_specs=[
        pl.BlockSpec((2, R, H), lambda i: (0, i, 0)),
        pl.BlockSpec((R, G), lambda i: (i, 0)),
        pl.BlockSpec((R, D), lambda i: (i, 0)),
        pl.BlockSpec((D, G), lambda i: (0, 0)),
        pl.BlockSpec((1, G), lambda i: (0, 0)),
    ],
    out_specs=pl.BlockSpec((R, D), lambda i: (i, 0)),
    out_shape=jax.ShapeDtypeStruct((N, D), jnp.float32),
)


def _sc_scatter_body(m_hbm, src_hbm, dst_hbm, out_hbm,
                     src_v, dst_v, rows_v, stage_v, shared, sem):
    c = lax.axis_index("c")
    s = lax.axis_index("s")

    def zrow(i, carry):
        def zcol(j, carry2):
            stage_v[i, pl.ds(j * LANES, LANES)] = jnp.zeros(
                (LANES,), jnp.float32)
            return carry2
        return lax.fori_loop(0, H // LANES, zcol, carry)

    lax.fori_loop(0, ZR, zrow, 0)
    for kk in range(RPS // ZR):
        pltpu.sync_copy(stage_v, shared.at[pl.ds(s * RPS + kk * ZR, ZR)])
    plsc.subcore_barrier()

    coff = c * NP

    def edge_chunk(i, carry):
        base = s * EPW + i * K
        pltpu.sync_copy(src_hbm.at[pl.ds(base, K)], src_v)
        pltpu.sync_copy(dst_hbm.at[pl.ds(base, K)], dst_v)
        for j in range(K // LANES):
            sl = pl.ds(j * LANES, LANES)
            src_v[sl] = src_v[sl] + coff
        pltpu.async_copy(m_hbm.at[src_v], rows_v, sem).wait()
        pltpu.sync_copy(rows_v, shared.at[dst_v], add=True)
        return carry

    lax.fori_loop(0, NCHUNK, edge_chunk, 0)
    plsc.subcore_barrier()

    for kk in range(RPS // ZR):
        r0 = s * RPS + kk * ZR
        pltpu.sync_copy(shared.at[pl.ds(r0, ZR)], stage_v)
        pltpu.sync_copy(stage_v, out_hbm.at[pl.ds(coff + r0, ZR)])


@functools.lru_cache(maxsize=None)
def _get_sc_scatter():
    return pl.kernel(
        _sc_scatter_body,
        out_type=jax.ShapeDtypeStruct((2 * NP, H), jnp.float32),
        mesh=plsc.VectorSubcoreMesh(core_axis_name="c", subcore_axis_name="s"),
        scratch_types=[
            pltpu.VMEM((K,), jnp.int32),
            pltpu.VMEM((K,), jnp.int32),
            pltpu.VMEM((K, H), jnp.float32),
            pltpu.VMEM((ZR, H), jnp.float32),
            pltpu.VMEM_SHARED((NP, H), jnp.float32),
            pltpu.SemaphoreType.DMA,
        ],
    )


@jax.jit
def kernel(x, edge_index, W_lin, b_lin, weight, w_ih, w_hh, b_ih, b_hh):
    src = edge_index[0]
    dst = edge_index[1]
    whhT = w_hh.T
    wihT = w_ih.T
    bhh = b_hh.reshape(1, G)
    bih = b_ih.reshape(1, G)

    h = _lin(x, W_lin.T, b_lin.reshape(1, D))
    for i in range(NLAYERS):
        m_split, gh = _msg(h, weight[i], whhT, bhh)
        agg_flat = _get_sc_scatter()(m_split.reshape(2 * NP, H), src, dst)
        h = _gru(agg_flat.reshape(2, NP, H), gh, h, wihT, bih)
    return h
